# baseline (speedup 1.0000x reference)
import functools

import jax
import jax.numpy as jnp
from jax import lax
from jax.experimental import pallas as pl
from jax.experimental.pallas import tpu as pltpu
from jax.experimental.pallas import tpu_sc as plsc

_LANES = 128
_GROUP = 5
_NSCAT = 1
_FB = 128
_VL = 16


def _make_sc_segment_sum(N, D, S):
    NC, NS = 2, 16
    DH = D // NC
    NV = DH // _VL
    NIDX = N // _LANES
    NG = NIDX // _GROUP
    CR = _GROUP * _LANES
    SROWS = S // NS
    MAXG = -(-NG // NS)
    MAXI = MAXG * _GROUP
    NSTRIPES = (_GROUP - _NSCAT) * (_LANES // _VL)
    RB = _NSCAT * _LANES

    mesh = plsc.VectorSubcoreMesh(core_axis_name="c", subcore_axis_name="s")

    @functools.partial(
        pl.kernel,
        mesh=mesh,
        out_type=jax.ShapeDtypeStruct((S, D), jnp.float32),
        scratch_types=[
            pltpu.VMEM((MAXI, _LANES), jnp.int32),
            pltpu.VMEM((2, CR, DH), jnp.float32),
            pltpu.VMEM((_FB, DH), jnp.float32),
            pltpu.VMEM((_FB,), jnp.int32),
            pltpu.VMEM_SHARED((S, DH), jnp.float32),
            pltpu.SemaphoreType.DMA((2,)),
            pltpu.SemaphoreType.DMA((2,)),
        ],
        compiler_params=pltpu.CompilerParams(use_tc_tiling_on_sc=False),
    )
    def seg_sum(nf_hbm, batch_hbm, zeros_hbm, out_hbm,
                idx_v, rows_v, fb, fids, acc, sem_ld, sem_sc):
        c = lax.axis_index("c")
        s = lax.axis_index("s")
        c0 = c * DH
        zv = jnp.zeros((_VL,), jnp.float32)

        lo = s * NG // NS
        hi = (s + 1) * NG // NS

        pltpu.async_copy(batch_hbm.at[pl.ds(lo * _GROUP, MAXI)],
                         idx_v, sem_ld.at[0])

        pltpu.sync_copy(zeros_hbm, acc.at[pl.ds(s * SROWS, SROWS)])

        pltpu.make_async_copy(batch_hbm.at[pl.ds(0, MAXI)],
                              idx_v, sem_ld.at[0]).wait()
        plsc.subcore_barrier()

        def fire_load(g, b):
            pltpu.async_copy(nf_hbm.at[pl.ds(g * CR, CR), pl.ds(c0, DH)],
                             rows_v.at[b], sem_ld.at[b])

        def wait_load(b):
            pltpu.make_async_copy(nf_hbm.at[pl.ds(0, CR), pl.ds(0, DH)],
                                  rows_v.at[b], sem_ld.at[b]).wait()

        def fire_scatters(k, b):
            for j in range(_NSCAT):
                pltpu.async_copy(rows_v.at[b, pl.ds(j * _LANES, _LANES)],
                                 acc.at[idx_v.at[k * _GROUP + j]],
                                 sem_sc.at[b], add=True)

        def drain_scatters(b):
            pltpu.make_async_copy(nf_hbm.at[pl.ds(0, _NSCAT * _LANES),
                                            pl.ds(0, DH)],
                                  rows_v.at[b, pl.ds(0, _NSCAT * _LANES)],
                                  sem_sc.at[b]).wait()


        lane = lax.iota(jnp.int32, _VL)

        def set_fid(p, cur):
            ch0 = (p // _VL) * _VL
            old = fids[pl.ds(ch0, _VL)]
            fids[pl.ds(ch0, _VL)] = jnp.where(lane == p - ch0, cur, old)

        def advance(new_id, cur, p):
            changed = new_id != cur
            p1 = p + jnp.where(changed & (cur >= 0), 1, 0).astype(jnp.int32)

            @pl.when(p1 == _FB)
            def _():
                pltpu.sync_copy(fb, acc.at[fids], add=True)

            p2 = lax.select(p1 == _FB, jnp.int32(0), p1)
            cur2 = lax.select(changed, new_id, cur)
            set_fid(p2, cur2)
            return changed, cur2, p2

        def load_open(changed, p):
            scale = lax.select(changed, jnp.float32(0), jnp.float32(1))
            return tuple(fb[p, pl.ds(m * _VL, _VL)] * scale
                         for m in range(NV))

        def store_open(p, avs):
            for m in range(NV):
                fb[p, pl.ds(m * _VL, _VL)] = avs[m]

        def accum_row(b, row, avs):
            return tuple(avs[m] + rows_v[b, row, pl.ds(m * _VL, _VL)]
                         for m in range(NV))

        def tec_stripe(q, k, b, cur, p):
            idxrow = k * _GROUP + _NSCAT + q // (_LANES // _VL)
            col0 = (q % (_LANES // _VL)) * _VL
            row0 = RB + q * _VL
            ids16 = idx_v[idxrow, pl.ds(col0, _VL)]
            id_f = ids16[0]
            id_l = ids16[_VL - 1]

            def uniform_stripe(cur, p):
                changed, cur, p = advance(id_f, cur, p)
                avs = load_open(changed, p)
                for t in range(_VL):
                    avs = accum_row(b, row0 + t, avs)
                store_open(p, avs)
                return cur, p

            def mixed_stripe(cur, p):
                for t in range(_VL):
                    changed, cur, p = advance(ids16[t], cur, p)
                    avs = load_open(changed, p)
                    avs = accum_row(b, row0 + t, avs)
                    store_open(p, avs)
                return cur, p

            return lax.cond(id_f == id_l, uniform_stripe, mixed_stripe,
                            cur, p)


        fire_load(lo, 0)

        def body(i, carry):
            cur, p = carry
            k = i - lo
            b = lax.rem(k, 2)
            wait_load(b)
            fire_scatters(k, b)

            @pl.when(i + 1 < hi)
            def _():
                @pl.when(k >= 1)
                def _():
                    drain_scatters(1 - b)
                fire_load(i + 1, 1 - b)

            def stripe_body(q, st):
                return tec_stripe(q, k, b, st[0], st[1])

            return lax.fori_loop(0, NSTRIPES, stripe_body, (cur, p))

        cur, p = lax.fori_loop(lo, hi, body, (jnp.int32(-1), jnp.int32(0)))

        def pad_body(i, carry):
            for m in range(NV):
                fb[i, pl.ds(m * _VL, _VL)] = zv
            return carry

        lax.fori_loop(p + 1, _FB, pad_body, 0)
        for w in range(_FB // _VL):
            old = fids[pl.ds(w * _VL, _VL)]
            fids[pl.ds(w * _VL, _VL)] = jnp.where(lane + w * _VL >= p + 1,
                                                  0, old)
        pltpu.sync_copy(fb, acc.at[fids], add=True)

        drain_scatters(0)
        drain_scatters(1)
        plsc.subcore_barrier()

        row0 = s * SROWS
        pltpu.sync_copy(acc.at[pl.ds(row0, SROWS)],
                        rows_v.at[0, pl.ds(0, SROWS)])
        pltpu.sync_copy(rows_v.at[0, pl.ds(0, SROWS)],
                        out_hbm.at[pl.ds(row0, SROWS), pl.ds(c0, DH)])

    return seg_sum


def kernel(node_features, batch, ptr):
    N, D = node_features.shape
    S = ptr.shape[0] - 1
    batch2d = batch.reshape(N // _LANES, _LANES)
    zeros = jnp.zeros((S // 16, D // 2), jnp.float32)
    f = _make_sc_segment_sum(N, D, S)
    return f(node_features, batch2d, zeros)


# speedup vs baseline: 1.1485x; 1.1485x over previous
import functools

import jax
import jax.numpy as jnp
from jax import lax
from jax.experimental import pallas as pl
from jax.experimental.pallas import tpu as pltpu
from jax.experimental.pallas import tpu_sc as plsc

_LANES = 128
_GROUP = 5
_NSCAT = 2
_FB = 128
_VL = 16


def _make_sc_segment_sum(N, D, S):
    NC, NS = 2, 16
    DH = D // NC
    NV = DH // _VL
    NIDX = N // _LANES
    NG = NIDX // _GROUP
    CR = _GROUP * _LANES
    SROWS = S // NS
    MAXG = -(-NG // NS)
    MAXI = MAXG * _GROUP
    NSTRIPES = (_GROUP - _NSCAT) * (_LANES // _VL)
    RB = _NSCAT * _LANES

    mesh = plsc.VectorSubcoreMesh(core_axis_name="c", subcore_axis_name="s")

    @functools.partial(
        pl.kernel,
        mesh=mesh,
        out_type=jax.ShapeDtypeStruct((S, D), jnp.float32),
        scratch_types=[
            pltpu.VMEM((MAXI, _LANES), jnp.int32),
            pltpu.VMEM((2, CR, DH), jnp.float32),
            pltpu.VMEM((_FB, DH), jnp.float32),
            pltpu.VMEM((_FB,), jnp.int32),
            pltpu.VMEM_SHARED((S, DH), jnp.float32),
            pltpu.SemaphoreType.DMA((2,)),
            pltpu.SemaphoreType.DMA((2,)),
        ],
        compiler_params=pltpu.CompilerParams(use_tc_tiling_on_sc=False),
    )
    def seg_sum(nf_hbm, batch_hbm, zeros_hbm, out_hbm,
                idx_v, rows_v, fb, fids, acc, sem_ld, sem_sc):
        c = lax.axis_index("c")
        s = lax.axis_index("s")
        c0 = c * DH
        zv = jnp.zeros((_VL,), jnp.float32)

        lo = s * NG // NS
        hi = (s + 1) * NG // NS

        pltpu.async_copy(batch_hbm.at[pl.ds(lo * _GROUP, MAXI)],
                         idx_v, sem_ld.at[0])

        pltpu.sync_copy(zeros_hbm, acc.at[pl.ds(s * SROWS, SROWS)])

        pltpu.make_async_copy(batch_hbm.at[pl.ds(0, MAXI)],
                              idx_v, sem_ld.at[0]).wait()
        plsc.subcore_barrier()

        def fire_load(g, b):
            pltpu.async_copy(nf_hbm.at[pl.ds(g * CR, CR), pl.ds(c0, DH)],
                             rows_v.at[b], sem_ld.at[b])

        def wait_load(b):
            pltpu.make_async_copy(nf_hbm.at[pl.ds(0, CR), pl.ds(0, DH)],
                                  rows_v.at[b], sem_ld.at[b]).wait()

        def fire_scatters(k, b):
            for j in range(_NSCAT):
                pltpu.async_copy(rows_v.at[b, pl.ds(j * _LANES, _LANES)],
                                 acc.at[idx_v.at[k * _GROUP + j]],
                                 sem_sc.at[b], add=True)

        def drain_scatters(b):
            pltpu.make_async_copy(nf_hbm.at[pl.ds(0, _NSCAT * _LANES),
                                            pl.ds(0, DH)],
                                  rows_v.at[b, pl.ds(0, _NSCAT * _LANES)],
                                  sem_sc.at[b]).wait()


        lane = lax.iota(jnp.int32, _VL)

        def set_fid(p, cur):
            ch0 = (p // _VL) * _VL
            old = fids[pl.ds(ch0, _VL)]
            fids[pl.ds(ch0, _VL)] = jnp.where(lane == p - ch0, cur, old)

        def advance(new_id, cur, p):
            changed = new_id != cur
            p1 = p + jnp.where(changed & (cur >= 0), 1, 0).astype(jnp.int32)

            @pl.when(p1 == _FB)
            def _():
                pltpu.sync_copy(fb, acc.at[fids], add=True)

            p2 = lax.select(p1 == _FB, jnp.int32(0), p1)
            cur2 = lax.select(changed, new_id, cur)

            @pl.when(changed)
            def _():
                set_fid(p2, cur2)

            return changed, cur2, p2

        def load_open(changed, p):
            scale = lax.select(changed, jnp.float32(0), jnp.float32(1))
            return tuple(fb[p, pl.ds(m * _VL, _VL)] * scale
                         for m in range(NV))

        def store_open(p, avs):
            for m in range(NV):
                fb[p, pl.ds(m * _VL, _VL)] = avs[m]

        def accum_row(b, row, avs):
            return tuple(avs[m] + rows_v[b, row, pl.ds(m * _VL, _VL)]
                         for m in range(NV))

        def tec_stripe(q, k, b, cur, p):
            idxrow = k * _GROUP + _NSCAT + q // (_LANES // _VL)
            col0 = (q % (_LANES // _VL)) * _VL
            row0 = RB + q * _VL
            ids16 = idx_v[idxrow, pl.ds(col0, _VL)]
            id_f = ids16[0]
            id_l = ids16[_VL - 1]

            def uniform_stripe(cur, p):
                changed, cur, p = advance(id_f, cur, p)
                avs = load_open(changed, p)
                for t in range(_VL):
                    avs = accum_row(b, row0 + t, avs)
                store_open(p, avs)
                return cur, p

            def mixed_stripe(cur, p):
                for t in range(_VL):
                    changed, cur, p = advance(ids16[t], cur, p)
                    avs = load_open(changed, p)
                    avs = accum_row(b, row0 + t, avs)
                    store_open(p, avs)
                return cur, p

            return lax.cond(id_f == id_l, uniform_stripe, mixed_stripe,
                            cur, p)


        fire_load(lo, 0)

        def body(i, carry):
            cur, p = carry
            k = i - lo
            b = lax.rem(k, 2)
            wait_load(b)

            @pl.when(i + 1 < hi)
            def _():
                @pl.when(k >= 1)
                def _():
                    drain_scatters(1 - b)
                fire_load(i + 1, 1 - b)

            fire_scatters(k, b)

            def stripe_body(q, st):
                return tec_stripe(q, k, b, st[0], st[1])

            return lax.fori_loop(0, NSTRIPES, stripe_body, (cur, p))

        cur, p = lax.fori_loop(lo, hi, body, (jnp.int32(-1), jnp.int32(0)))

        def pad_body(i, carry):
            for m in range(NV):
                fb[i, pl.ds(m * _VL, _VL)] = zv
            return carry

        lax.fori_loop(p + 1, _FB, pad_body, 0)
        for w in range(_FB // _VL):
            old = fids[pl.ds(w * _VL, _VL)]
            fids[pl.ds(w * _VL, _VL)] = jnp.where(lane + w * _VL >= p + 1,
                                                  0, old)
        pltpu.sync_copy(fb, acc.at[fids], add=True)

        drain_scatters(0)
        drain_scatters(1)
        plsc.subcore_barrier()

        row0 = s * SROWS
        pltpu.sync_copy(acc.at[pl.ds(row0, SROWS)],
                        rows_v.at[0, pl.ds(0, SROWS)])
        pltpu.sync_copy(rows_v.at[0, pl.ds(0, SROWS)],
                        out_hbm.at[pl.ds(row0, SROWS), pl.ds(c0, DH)])

    return seg_sum


def kernel(node_features, batch, ptr):
    N, D = node_features.shape
    S = ptr.shape[0] - 1
    batch2d = batch.reshape(N // _LANES, _LANES)
    zeros = jnp.zeros((S // 16, D // 2), jnp.float32)
    f = _make_sc_segment_sum(N, D, S)
    return f(node_features, batch2d, zeros)
